# baseline (speedup 1.0000x reference)
import functools

import jax
import jax.numpy as jnp
from jax import lax
from jax.experimental import pallas as pl
from jax.experimental.pallas import tpu as pltpu
import jax.experimental.pallas.tpu_sc as plsc

B = 4096
L = 50
D = 128
HID = 128
NCLS = 1000
THRESH = 0.15

NC, NS = 2, 16
NW = NC * NS
NCHUNK = 4
PB = B // NCHUNK
BPW = PB // NW
BPC = 2
ROWS = BPC * L
CHUNKS = BPW // BPC
NLANE = 8

MLP_BLK = 512


def _make_pool_body(piece):
    base_row = piece * (PB // BPC)

    def _pool_body(x_hbm, table_hbm, out_hbm, idx_v, rows_v, acc_v, sem0, sem1):
        cid = lax.axis_index("c")
        sid = lax.axis_index("s")
        wid = sid * NC + cid

        pltpu.sync_copy(
            x_hbm.at[pl.ds(base_row + wid * CHUNKS, CHUNKS)], idx_v)

        sems = (sem0, sem1)

        def start_gather(chunk, buf):
            chunk = jnp.minimum(chunk, CHUNKS - 1)
            pltpu.async_copy(
                table_hbm.at[idx_v.at[chunk]], rows_v.at[buf], sems[buf])

        def wait_gather(buf):
            pltpu.make_async_copy(
                table_hbm.at[idx_v.at[0]], rows_v.at[buf], sems[buf]).wait()

        def reduce_chunk(chunk, buf):
            def body(r, accs):
                new = []
                for j in range(BPC):
                    for k in range(NLANE):
                        new.append(accs[j * NLANE + k]
                                   + rows_v[buf, j * L + r, pl.ds(k * 16, 16)])
                return tuple(new)

            init = tuple(jnp.zeros((16,), jnp.float32)
                         for _ in range(BPC * NLANE))
            accs = lax.fori_loop(0, L, body, init)
            for j in range(BPC):
                row = chunk * BPC + j
                for k in range(NLANE):
                    acc_v[row, pl.ds(k * 16, 16)] = accs[j * NLANE + k]

        start_gather(jnp.int32(0), 0)

        def outer(g, _):
            c0 = g * 2
            start_gather(c0 + 1, 1)
            wait_gather(0)
            reduce_chunk(c0, 0)
            start_gather(c0 + 2, 0)
            wait_gather(1)
            reduce_chunk(c0 + 1, 1)
            return 0

        lax.fori_loop(0, CHUNKS // 2, outer, 0)
        wait_gather(0)

        pltpu.sync_copy(acc_v, out_hbm.at[pl.ds(wid * BPW, BPW)])

    return _pool_body


def _pool(piece, x2, emb_table):
    mesh = plsc.VectorSubcoreMesh(core_axis_name="c", subcore_axis_name="s",
                                  num_cores=NC, num_subcores=NS)
    return pl.kernel(
        _make_pool_body(piece),
        out_type=jax.ShapeDtypeStruct((PB, D), jnp.float32),
        mesh=mesh,
        scratch_types=[
            pltpu.VMEM((CHUNKS, ROWS), jnp.int32),
            pltpu.VMEM((2, ROWS, D), jnp.float32),
            pltpu.VMEM((BPW, D), jnp.float32),
            pltpu.SemaphoreType.DMA,
            pltpu.SemaphoreType.DMA,
        ],
        name=f"pool_p{piece}",
    )(x2, emb_table)


def _mlp_first_body(x_ref, w1_ref, b1_ref, w2_ref, b2_ref, o_ref):
    xm = x_ref[...] / jnp.float32(L)
    h = jnp.dot(xm, w1_ref[...], preferred_element_type=jnp.float32)
    h = h + b1_ref[...]
    h = jnp.where(h >= THRESH, h, 0.0)
    o_ref[...] = (jnp.dot(h, w2_ref[...], preferred_element_type=jnp.float32)
                  + b2_ref[...])


def _mlp_alias_body(prev_ref, x_ref, w1_ref, b1_ref, w2_ref, b2_ref, o_ref):
    del prev_ref
    _mlp_first_body(x_ref, w1_ref, b1_ref, w2_ref, b2_ref, o_ref)


def _mlp_piece(piece, out_prev, pooled, w1, b1, w2, b2):
    grid = (PB // MLP_BLK,)
    row0 = piece * (PB // MLP_BLK)
    out_spec = pl.BlockSpec((MLP_BLK, NCLS), lambda i: (row0 + i, 0))
    data_specs = [
        pl.BlockSpec((MLP_BLK, D), lambda i: (i, 0)),
        pl.BlockSpec((D, HID), lambda i: (0, 0)),
        pl.BlockSpec((1, HID), lambda i: (0, 0)),
        pl.BlockSpec((HID, NCLS), lambda i: (0, 0)),
        pl.BlockSpec((1, NCLS), lambda i: (0, 0)),
    ]
    out_shape = jax.ShapeDtypeStruct((B, NCLS), jnp.float32)
    if piece == 0:
        return pl.pallas_call(
            _mlp_first_body,
            grid=grid,
            in_specs=data_specs,
            out_specs=out_spec,
            out_shape=out_shape,
            name="mlp_p0",
        )(pooled, w1, b1, w2, b2)
    return pl.pallas_call(
        _mlp_alias_body,
        grid=grid,
        in_specs=[pl.BlockSpec((8, 128), lambda i: (0, 0))] + data_specs,
        out_specs=out_spec,
        out_shape=out_shape,
        input_output_aliases={0: 0},
        name=f"mlp_p{piece}",
    )(out_prev, pooled, w1, b1, w2, b2)


def kernel(x, emb_table, W1, b1, W2, b2):
    x2 = x.reshape(B * L // ROWS, ROWS)
    b1r = b1.reshape(1, HID)
    b2r = b2.reshape(1, NCLS)
    pooled = [_pool(i, x2, emb_table) for i in range(NCHUNK)]
    out = None
    for i in range(NCHUNK):
        out = _mlp_piece(i, out, pooled[i], W1, b1r, W2, b2r)
    return out


# speedup vs baseline: 1.3442x; 1.3442x over previous
import functools

import jax
import jax.numpy as jnp
from jax import lax
from jax.experimental import pallas as pl
from jax.experimental.pallas import tpu as pltpu
import jax.experimental.pallas.tpu_sc as plsc

B = 4096
L = 50
D = 128
HID = 128
NCLS = 1000
THRESH = 0.15

NC, NS = 2, 16
NW = NC * NS
BPW = B // NW
BPC = 2
ROWS = BPC * L
CHUNKS = BPW // BPC
NLANE = 8


def _pool_body(x_hbm, table_hbm, out_hbm, idx_v, rows_v, acc_v, sem0, sem1):
    cid = lax.axis_index("c")
    sid = lax.axis_index("s")
    wid = sid * NC + cid

    pltpu.sync_copy(x_hbm.at[pl.ds(wid * CHUNKS, CHUNKS)], idx_v)

    sems = (sem0, sem1)

    def start_gather(chunk, buf):
        chunk = jnp.minimum(chunk, CHUNKS - 1)
        pltpu.async_copy(
            table_hbm.at[idx_v.at[chunk]], rows_v.at[buf], sems[buf])

    def wait_gather(buf):
        pltpu.make_async_copy(
            table_hbm.at[idx_v.at[0]], rows_v.at[buf], sems[buf]).wait()

    def reduce_chunk(chunk, buf):
        def body(r, accs):
            new = []
            for j in range(BPC):
                for k in range(NLANE):
                    new.append(accs[j * NLANE + k]
                               + rows_v[buf, j * L + r, pl.ds(k * 16, 16)])
            return tuple(new)

        init = tuple(jnp.zeros((16,), jnp.float32) for _ in range(BPC * NLANE))
        accs = lax.fori_loop(0, L, body, init)
        for j in range(BPC):
            row = chunk * BPC + j
            for k in range(NLANE):
                acc_v[row, pl.ds(k * 16, 16)] = accs[j * NLANE + k]

    start_gather(jnp.int32(0), 0)

    def outer(g, _):
        c0 = g * 2
        start_gather(c0 + 1, 1)
        wait_gather(0)
        reduce_chunk(c0, 0)
        start_gather(c0 + 2, 0)
        wait_gather(1)
        reduce_chunk(c0 + 1, 1)
        return 0

    lax.fori_loop(0, CHUNKS // 2, outer, 0)
    wait_gather(0)

    pltpu.sync_copy(acc_v, out_hbm.at[pl.ds(wid * BPW, BPW)])


@functools.partial(jax.jit, static_argnames=())
def _pool(x2, emb_table):
    mesh = plsc.VectorSubcoreMesh(core_axis_name="c", subcore_axis_name="s",
                                  num_cores=NC, num_subcores=NS)
    return pl.kernel(
        _pool_body,
        out_type=jax.ShapeDtypeStruct((B, D), jnp.float32),
        mesh=mesh,
        scratch_types=[
            pltpu.VMEM((CHUNKS, ROWS), jnp.int32),
            pltpu.VMEM((2, ROWS, D), jnp.float32),
            pltpu.VMEM((BPW, D), jnp.float32),
            pltpu.SemaphoreType.DMA,
            pltpu.SemaphoreType.DMA,
        ],
    )(x2, emb_table)


def _mlp_body(x_ref, w1_ref, b1_ref, w2t_ref, b2t_ref, ot_ref):
    xm = x_ref[...] / jnp.float32(L)
    h = jnp.dot(xm, w1_ref[...], preferred_element_type=jnp.float32)
    h = h + b1_ref[...]
    h = jnp.where(h >= THRESH, h, 0.0)
    ot = lax.dot_general(w2t_ref[...], h, (((1,), (1,)), ((), ())),
                         preferred_element_type=jnp.float32)
    ot_ref[...] = ot + b2t_ref[...]


def _mlp(pooled, w1, b1, w2t, b2t):
    blk = 512
    return pl.pallas_call(
        _mlp_body,
        grid=(B // blk,),
        in_specs=[
            pl.BlockSpec((blk, D), lambda i: (i, 0)),
            pl.BlockSpec((D, HID), lambda i: (0, 0)),
            pl.BlockSpec((1, HID), lambda i: (0, 0)),
            pl.BlockSpec((NCLS, HID), lambda i: (0, 0)),
            pl.BlockSpec((NCLS, 1), lambda i: (0, 0)),
        ],
        out_specs=pl.BlockSpec((NCLS, blk), lambda i: (0, i)),
        out_shape=jax.ShapeDtypeStruct((NCLS, B), jnp.float32),
    )(pooled, w1, b1, w2t, b2t)


def kernel(x, emb_table, W1, b1, W2, b2):
    x2 = x.reshape(NW * CHUNKS, ROWS)
    pooled = _pool(x2, emb_table)
    out_t = _mlp(pooled, W1, b1.reshape(1, HID), W2.T, b2.reshape(NCLS, 1))
    return out_t.T
